# Initial kernel scaffold; baseline (speedup 1.0000x reference)
import jax
import jax.numpy as jnp
from jax.experimental import pallas as pl


def kernel(x, edge_index, edge_attr, batch, W_edge, eps, W1, b1, W2, b2, Wv1, bv1, Wv2, bv2, vn0, Wpred, bpred):
    raise NotImplementedError("write your pallas kernel here")


# speedup vs baseline: 1.5824x; 1.5824x over previous
import functools

import jax
import jax.numpy as jnp
from jax import lax
from jax.experimental import pallas as pl
from jax.experimental.pallas import tpu as pltpu
from jax.experimental.pallas import tpu_sc as plsc

N = 10000
E = 320000
D = 128
DE = 16
L = 5
G = 64
NCLS = 10

NP = 10240
NB = 2048
NCORE = 2
NSUB = 16
NW = NCORE * NSUB
CH = 64
KCH = 160
GK = 16
NGRP = KCH // GK
EPT = CH * KCH
EP = EPT * NW
EB = 8192
RPT = NP // NSUB

_f32 = jnp.float32


def _emb_body(ea_ref, w_ref, out_ref):
    out_ref[0] = lax.dot_general(
        ea_ref[...], w_ref[0], (((1,), (0,)), ((), ())),
        preferred_element_type=_f32)


_emb_call = pl.pallas_call(
    _emb_body,
    grid=(L, EP // EB),
    in_specs=[
        pl.BlockSpec((EB, DE), lambda l, e: (e, 0)),
        pl.BlockSpec((1, DE, D), lambda l, e: (l, 0, 0)),
    ],
    out_specs=pl.BlockSpec((1, EB, D), lambda l, e: (l, e, 0)),
    out_shape=jax.ShapeDtypeStruct((L, EP, D), _f32),
)


def _layer_in_body(h_ref, b_ref, vn_ref, wv1_ref, bv1_ref, wv2_ref, bv2_ref,
                   hl_ref, vnn_ref, acc_ref):
    i = pl.program_id(0)
    nsteps = pl.num_programs(0)
    brow = b_ref[0:1, pl.ds(i * NB, NB)]
    onehot = (lax.broadcasted_iota(jnp.int32, (G, NB), 0) == brow).astype(_f32)
    vnb = lax.dot_general(onehot, vn_ref[...], (((0,), (0,)), ((), ())),
                          preferred_element_type=_f32)
    hl = h_ref[...] + vnb
    hl_ref[...] = hl

    @pl.when(i == 0)
    def _():
        acc_ref[...] = jnp.zeros_like(acc_ref)

    acc_ref[...] += lax.dot_general(onehot, hl, (((1,), (0,)), ((), ())),
                                    preferred_element_type=_f32)

    @pl.when(i == nsteps - 1)
    def _():
        vt = acc_ref[...] + vn_ref[...]
        u = jnp.maximum(
            lax.dot_general(vt, wv1_ref[...], (((1,), (0,)), ((), ())),
                            preferred_element_type=_f32) + bv1_ref[...], 0.0)
        vnn = jnp.maximum(
            lax.dot_general(u, wv2_ref[...], (((1,), (0,)), ((), ())),
                            preferred_element_type=_f32) + bv2_ref[...], 0.0)
        vnn_ref[...] = vnn


_layer_in_call = pl.pallas_call(
    _layer_in_body,
    grid=(NP // NB,),
    in_specs=[
        pl.BlockSpec((NB, D), lambda i: (i, 0)),
        pl.BlockSpec((1, NP), lambda i: (0, 0)),
        pl.BlockSpec((G, D), lambda i: (0, 0)),
        pl.BlockSpec((D, 2 * D), lambda i: (0, 0)),
        pl.BlockSpec((1, 2 * D), lambda i: (0, 0)),
        pl.BlockSpec((2 * D, D), lambda i: (0, 0)),
        pl.BlockSpec((1, D), lambda i: (0, 0)),
    ],
    out_specs=[
        pl.BlockSpec((NB, D), lambda i: (i, 0)),
        pl.BlockSpec((G, D), lambda i: (0, 0)),
    ],
    out_shape=[
        jax.ShapeDtypeStruct((NP, D), _f32),
        jax.ShapeDtypeStruct((G, D), _f32),
    ],
    scratch_shapes=[pltpu.VMEM((G, D), _f32)],
)


def _layer_out_body(hl_ref, agg_ref, eps_ref, w1_ref, b1_ref, w2_ref, b2_ref,
                    out_ref, *, final):
    z = ((1.0 + eps_ref[0, 0]) * hl_ref[...] + agg_ref[0] + agg_ref[1])
    t = jnp.maximum(
        lax.dot_general(z, w1_ref[...], (((1,), (0,)), ((), ())),
                        preferred_element_type=_f32) + b1_ref[...], 0.0)
    h2 = lax.dot_general(t, w2_ref[...], (((1,), (0,)), ((), ())),
                         preferred_element_type=_f32) + b2_ref[...]
    if not final:
        h2 = jnp.maximum(h2, 0.0)
    out_ref[...] = h2


def _make_layer_out_call(final):
    return pl.pallas_call(
        functools.partial(_layer_out_body, final=final),
        grid=(NP // NB,),
        in_specs=[
            pl.BlockSpec((NB, D), lambda i: (i, 0)),
            pl.BlockSpec((2, NB, D), lambda i: (0, i, 0)),
            pl.BlockSpec((1, 1), lambda i: (0, 0)),
            pl.BlockSpec((D, 2 * D), lambda i: (0, 0)),
            pl.BlockSpec((1, 2 * D), lambda i: (0, 0)),
            pl.BlockSpec((2 * D, D), lambda i: (0, 0)),
            pl.BlockSpec((1, D), lambda i: (0, 0)),
        ],
        out_specs=pl.BlockSpec((NB, D), lambda i: (i, 0)),
        out_shape=jax.ShapeDtypeStruct((NP, D), _f32),
    )


_layer_out_call = _make_layer_out_call(False)
_layer_out_final_call = _make_layer_out_call(True)


def _head_body(h_ref, b_ref, wp_ref, bp_ref, out_ref, acc_ref, cnt_ref):
    i = pl.program_id(0)
    nsteps = pl.num_programs(0)
    brow = b_ref[0:1, pl.ds(i * NB, NB)]
    onehot = (lax.broadcasted_iota(jnp.int32, (G, NB), 0) == brow).astype(_f32)

    @pl.when(i == 0)
    def _():
        acc_ref[...] = jnp.zeros_like(acc_ref)
        cnt_ref[...] = jnp.zeros_like(cnt_ref)

    acc_ref[...] += lax.dot_general(onehot, h_ref[...], (((1,), (0,)), ((), ())),
                                    preferred_element_type=_f32)
    cnt_ref[...] += jnp.broadcast_to(
        jnp.sum(onehot, axis=1, keepdims=True), (G, D))

    @pl.when(i == nsteps - 1)
    def _():
        pool = acc_ref[...] / jnp.maximum(cnt_ref[...], 1.0)
        out_ref[...] = lax.dot_general(
            pool, wp_ref[...], (((1,), (0,)), ((), ())),
            preferred_element_type=_f32) + bp_ref[...]


_head_call = pl.pallas_call(
    _head_body,
    grid=(NP // NB,),
    in_specs=[
        pl.BlockSpec((NB, D), lambda i: (i, 0)),
        pl.BlockSpec((1, NP), lambda i: (0, 0)),
        pl.BlockSpec((D, NCLS), lambda i: (0, 0)),
        pl.BlockSpec((1, NCLS), lambda i: (0, 0)),
    ],
    out_specs=pl.BlockSpec((G, NCLS), lambda i: (0, 0)),
    out_shape=jax.ShapeDtypeStruct((G, NCLS), _f32),
    scratch_shapes=[pltpu.VMEM((G, D), _f32), pltpu.VMEM((G, D), _f32)],
)


def _sc_body(hl_hbm, src_hbm, dst_hbm, emb_hbm, zro_hbm, out_hbm,
             agg_sh, sidx, didx, embv, rowsv, sem, *, layer):
    c = lax.axis_index("c")
    s = lax.axis_index("s")
    wid = c * NSUB + s

    pltpu.sync_copy(zro_hbm, agg_sh.at[pl.ds(s * RPT, RPT)])
    plsc.subcore_barrier()

    def group(gi, carry0):
        pltpu.sync_copy(src_hbm.at[pl.ds(wid * KCH + gi * GK, GK)], sidx)
        pltpu.sync_copy(dst_hbm.at[pl.ds(wid * KCH + gi * GK, GK)], didx)

        def chunk(j, carry):
            cidx = gi * GK + j
            pltpu.sync_copy(emb_hbm.at[layer, pl.ds(wid * EPT + cidx * CH, CH)],
                            embv)
            pltpu.async_copy(hl_hbm.at[sidx.at[j]], rowsv, sem).wait()

            def vstep(r, carry2):
                for cc in range(D // 16):
                    o = cc * 16
                    v = rowsv[r, pl.ds(o, 16)] + embv[r, pl.ds(o, 16)]
                    embv[r, pl.ds(o, 16)] = jnp.maximum(v, 0.0)
                return carry2

            lax.fori_loop(0, CH, vstep, 0)
            pltpu.sync_copy(embv, agg_sh.at[didx.at[j]], add=True)
            return carry

        lax.fori_loop(0, GK, chunk, 0)
        return carry0

    lax.fori_loop(0, NGRP, group, 0)
    plsc.subcore_barrier()
    pltpu.sync_copy(agg_sh.at[pl.ds(s * RPT, RPT)],
                    out_hbm.at[c, pl.ds(s * RPT, RPT)])


def _make_sc_call(layer):
    return pl.kernel(
        functools.partial(_sc_body, layer=layer),
        out_type=jax.ShapeDtypeStruct((NCORE, NP, D), _f32),
        mesh=plsc.VectorSubcoreMesh(core_axis_name="c", subcore_axis_name="s",
                                    num_cores=NCORE, num_subcores=NSUB),
        scratch_types=[
            pltpu.VMEM_SHARED((NP, D), _f32),
            pltpu.VMEM((GK, CH), jnp.int32),
            pltpu.VMEM((GK, CH), jnp.int32),
            pltpu.VMEM((CH, D), _f32),
            pltpu.VMEM((CH, D), _f32),
            pltpu.SemaphoreType.DMA,
        ],
    )


_sc_calls = [_make_sc_call(l) for l in range(L)]


def kernel(x, edge_index, edge_attr, batch, W_edge, eps, W1, b1, W2, b2,
           Wv1, bv1, Wv2, bv2, vn0, Wpred, bpred):
    x_pad = jnp.zeros((NP, D), _f32).at[:N].set(x)
    batch_pad = jnp.full((1, NP), G, jnp.int32).at[0, :N].set(batch)
    src_pad = (jnp.zeros((EP,), jnp.int32).at[:E].set(edge_index[0])
               .reshape(EP // CH, CH))
    dst_pad = (jnp.full((EP,), N, jnp.int32).at[:E].set(edge_index[1])
               .reshape(EP // CH, CH))
    ea_pad = jnp.zeros((EP, DE), _f32).at[:E].set(edge_attr)
    zeros2d = jnp.zeros((RPT, D), _f32)
    vn = jnp.broadcast_to(vn0, (G, D)) + jnp.zeros((G, D), _f32)

    emb_all = _emb_call(ea_pad, W_edge)

    h = x_pad
    for l in range(L):
        lv = min(l, L - 2)
        hl, vn_next = _layer_in_call(h, batch_pad, vn, Wv1[lv],
                                     bv1[lv].reshape(1, 2 * D), Wv2[lv],
                                     bv2[lv].reshape(1, D))
        agg = _sc_calls[l](hl, src_pad, dst_pad, emb_all, zeros2d)
        call = _layer_out_final_call if l == L - 1 else _layer_out_call
        h = call(hl, agg, eps[l].reshape(1, 1), W1[l], b1[l].reshape(1, 2 * D),
                 W2[l], b2[l].reshape(1, D))
        vn = vn_next

    return _head_call(h, batch_pad, Wpred, bpred.reshape(1, NCLS))


# speedup vs baseline: 2.1876x; 1.3825x over previous
import functools

import jax
import jax.numpy as jnp
from jax import lax
from jax.experimental import pallas as pl
from jax.experimental.pallas import tpu as pltpu
from jax.experimental.pallas import tpu_sc as plsc

N = 10000
E = 320000
D = 128
DE = 16
L = 5
G = 64
NCLS = 10

NP = 10240
NB = 2048
NCORE = 2
NSUB = 16
NW = NCORE * NSUB
CH = 64
KCH = 160
GK = 32
NGRP = KCH // GK
EPT = CH * KCH
EP = EPT * NW
EB = 8192
RPT = NP // NSUB

_f32 = jnp.float32


def _emb_body(ea_ref, w_ref, out_ref):
    out_ref[0] = lax.dot_general(
        ea_ref[...], w_ref[0], (((1,), (0,)), ((), ())),
        preferred_element_type=_f32)


_emb_call = pl.pallas_call(
    _emb_body,
    grid=(L, EP // EB),
    in_specs=[
        pl.BlockSpec((EB, DE), lambda l, e: (e, 0)),
        pl.BlockSpec((1, DE, D), lambda l, e: (l, 0, 0)),
    ],
    out_specs=pl.BlockSpec((1, EB, D), lambda l, e: (l, e, 0)),
    out_shape=jax.ShapeDtypeStruct((L, EP, D), _f32),
)


def _layer_in_body(h_ref, b_ref, vn_ref, wv1_ref, bv1_ref, wv2_ref, bv2_ref,
                   hl_ref, vnn_ref, acc_ref):
    i = pl.program_id(0)
    nsteps = pl.num_programs(0)
    brow = b_ref[0:1, pl.ds(i * NB, NB)]
    onehot = (lax.broadcasted_iota(jnp.int32, (G, NB), 0) == brow).astype(_f32)
    vnb = lax.dot_general(onehot, vn_ref[...], (((0,), (0,)), ((), ())),
                          preferred_element_type=_f32)
    hl = h_ref[...] + vnb
    hl_ref[...] = hl

    @pl.when(i == 0)
    def _():
        acc_ref[...] = jnp.zeros_like(acc_ref)

    acc_ref[...] += lax.dot_general(onehot, hl, (((1,), (0,)), ((), ())),
                                    preferred_element_type=_f32)

    @pl.when(i == nsteps - 1)
    def _():
        vt = acc_ref[...] + vn_ref[...]
        u = jnp.maximum(
            lax.dot_general(vt, wv1_ref[...], (((1,), (0,)), ((), ())),
                            preferred_element_type=_f32) + bv1_ref[...], 0.0)
        vnn = jnp.maximum(
            lax.dot_general(u, wv2_ref[...], (((1,), (0,)), ((), ())),
                            preferred_element_type=_f32) + bv2_ref[...], 0.0)
        vnn_ref[...] = vnn


_layer_in_call = pl.pallas_call(
    _layer_in_body,
    grid=(NP // NB,),
    in_specs=[
        pl.BlockSpec((NB, D), lambda i: (i, 0)),
        pl.BlockSpec((1, NP), lambda i: (0, 0)),
        pl.BlockSpec((G, D), lambda i: (0, 0)),
        pl.BlockSpec((D, 2 * D), lambda i: (0, 0)),
        pl.BlockSpec((1, 2 * D), lambda i: (0, 0)),
        pl.BlockSpec((2 * D, D), lambda i: (0, 0)),
        pl.BlockSpec((1, D), lambda i: (0, 0)),
    ],
    out_specs=[
        pl.BlockSpec((NB, D), lambda i: (i, 0)),
        pl.BlockSpec((G, D), lambda i: (0, 0)),
    ],
    out_shape=[
        jax.ShapeDtypeStruct((NP, D), _f32),
        jax.ShapeDtypeStruct((G, D), _f32),
    ],
    scratch_shapes=[pltpu.VMEM((G, D), _f32)],
)


def _layer_out_body(hl_ref, agg_ref, eps_ref, w1_ref, b1_ref, w2_ref, b2_ref,
                    out_ref, *, final):
    z = ((1.0 + eps_ref[0, 0]) * hl_ref[...] + agg_ref[0] + agg_ref[1])
    t = jnp.maximum(
        lax.dot_general(z, w1_ref[...], (((1,), (0,)), ((), ())),
                        preferred_element_type=_f32) + b1_ref[...], 0.0)
    h2 = lax.dot_general(t, w2_ref[...], (((1,), (0,)), ((), ())),
                         preferred_element_type=_f32) + b2_ref[...]
    if not final:
        h2 = jnp.maximum(h2, 0.0)
    out_ref[...] = h2


def _make_layer_out_call(final):
    return pl.pallas_call(
        functools.partial(_layer_out_body, final=final),
        grid=(NP // NB,),
        in_specs=[
            pl.BlockSpec((NB, D), lambda i: (i, 0)),
            pl.BlockSpec((2, NB, D), lambda i: (0, i, 0)),
            pl.BlockSpec((1, 1), lambda i: (0, 0)),
            pl.BlockSpec((D, 2 * D), lambda i: (0, 0)),
            pl.BlockSpec((1, 2 * D), lambda i: (0, 0)),
            pl.BlockSpec((2 * D, D), lambda i: (0, 0)),
            pl.BlockSpec((1, D), lambda i: (0, 0)),
        ],
        out_specs=pl.BlockSpec((NB, D), lambda i: (i, 0)),
        out_shape=jax.ShapeDtypeStruct((NP, D), _f32),
    )


_layer_out_call = _make_layer_out_call(False)
_layer_out_final_call = _make_layer_out_call(True)


def _head_body(h_ref, b_ref, wp_ref, bp_ref, out_ref, acc_ref, cnt_ref):
    i = pl.program_id(0)
    nsteps = pl.num_programs(0)
    brow = b_ref[0:1, pl.ds(i * NB, NB)]
    onehot = (lax.broadcasted_iota(jnp.int32, (G, NB), 0) == brow).astype(_f32)

    @pl.when(i == 0)
    def _():
        acc_ref[...] = jnp.zeros_like(acc_ref)
        cnt_ref[...] = jnp.zeros_like(cnt_ref)

    acc_ref[...] += lax.dot_general(onehot, h_ref[...], (((1,), (0,)), ((), ())),
                                    preferred_element_type=_f32)
    cnt_ref[...] += jnp.broadcast_to(
        jnp.sum(onehot, axis=1, keepdims=True), (G, D))

    @pl.when(i == nsteps - 1)
    def _():
        pool = acc_ref[...] / jnp.maximum(cnt_ref[...], 1.0)
        out_ref[...] = lax.dot_general(
            pool, wp_ref[...], (((1,), (0,)), ((), ())),
            preferred_element_type=_f32) + bp_ref[...]


_head_call = pl.pallas_call(
    _head_body,
    grid=(NP // NB,),
    in_specs=[
        pl.BlockSpec((NB, D), lambda i: (i, 0)),
        pl.BlockSpec((1, NP), lambda i: (0, 0)),
        pl.BlockSpec((D, NCLS), lambda i: (0, 0)),
        pl.BlockSpec((1, NCLS), lambda i: (0, 0)),
    ],
    out_specs=pl.BlockSpec((G, NCLS), lambda i: (0, 0)),
    out_shape=jax.ShapeDtypeStruct((G, NCLS), _f32),
    scratch_shapes=[pltpu.VMEM((G, D), _f32), pltpu.VMEM((G, D), _f32)],
)


def _sc_body(hl_hbm, src_hbm, dst_hbm, emb_hbm, zro_hbm, out_hbm,
             agg_sh, sidx, didx, e0, e1, r0, r1,
             esem0, esem1, gsem0, gsem1, *, layer):
    c = lax.axis_index("c")
    s = lax.axis_index("s")
    wid = c * NSUB + s

    pltpu.sync_copy(zro_hbm, agg_sh.at[pl.ds(s * RPT, RPT)])
    plsc.subcore_barrier()

    def emb_src(cidx):
        return emb_hbm.at[layer, pl.ds(wid * EPT + cidx * CH, CH)]

    def start(cidx, j, ebuf, rbuf, esem, gsem):
        pltpu.async_copy(emb_src(cidx), ebuf, esem)
        pltpu.async_copy(hl_hbm.at[sidx.at[j]], rbuf, gsem)

    def finish(cidx, j, ebuf, rbuf, esem, gsem):
        pltpu.make_async_copy(emb_src(cidx), ebuf, esem).wait()
        pltpu.make_async_copy(hl_hbm.at[sidx.at[j]], rbuf, gsem).wait()

        def vstep(r, carry2):
            for cc in range(D // 16):
                o = cc * 16
                v = rbuf[r, pl.ds(o, 16)] + ebuf[r, pl.ds(o, 16)]
                ebuf[r, pl.ds(o, 16)] = jnp.maximum(v, 0.0)
            return carry2

        lax.fori_loop(0, CH, vstep, 0)
        pltpu.sync_copy(ebuf, agg_sh.at[didx.at[j]], add=True)

    def group(gi, carry0):
        g0 = gi * GK
        pltpu.sync_copy(src_hbm.at[pl.ds(wid * KCH + g0, GK)], sidx)
        pltpu.sync_copy(dst_hbm.at[pl.ds(wid * KCH + g0, GK)], didx)
        start(g0, 0, e0, r0, esem0, gsem0)

        def pair(pj, carry):
            cb = g0 + 2 * pj
            start(cb + 1, 2 * pj + 1, e1, r1, esem1, gsem1)
            finish(cb, 2 * pj, e0, r0, esem0, gsem0)
            start(cb + 2, 2 * pj + 2, e0, r0, esem0, gsem0)
            finish(cb + 1, 2 * pj + 1, e1, r1, esem1, gsem1)
            return carry

        lax.fori_loop(0, GK // 2 - 1, pair, 0)
        start(g0 + GK - 1, GK - 1, e1, r1, esem1, gsem1)
        finish(g0 + GK - 2, GK - 2, e0, r0, esem0, gsem0)
        finish(g0 + GK - 1, GK - 1, e1, r1, esem1, gsem1)
        return carry0

    lax.fori_loop(0, NGRP, group, 0)
    plsc.subcore_barrier()
    pltpu.sync_copy(agg_sh.at[pl.ds(s * RPT, RPT)],
                    out_hbm.at[c, pl.ds(s * RPT, RPT)])


def _make_sc_call(layer):
    return pl.kernel(
        functools.partial(_sc_body, layer=layer),
        out_type=jax.ShapeDtypeStruct((NCORE, NP, D), _f32),
        mesh=plsc.VectorSubcoreMesh(core_axis_name="c", subcore_axis_name="s",
                                    num_cores=NCORE, num_subcores=NSUB),
        scratch_types=[
            pltpu.VMEM_SHARED((NP, D), _f32),
            pltpu.VMEM((GK, CH), jnp.int32),
            pltpu.VMEM((GK, CH), jnp.int32),
            pltpu.VMEM((CH, D), _f32),
            pltpu.VMEM((CH, D), _f32),
            pltpu.VMEM((CH, D), _f32),
            pltpu.VMEM((CH, D), _f32),
            pltpu.SemaphoreType.DMA,
            pltpu.SemaphoreType.DMA,
            pltpu.SemaphoreType.DMA,
            pltpu.SemaphoreType.DMA,
        ],
    )


_sc_calls = [_make_sc_call(l) for l in range(L)]


def kernel(x, edge_index, edge_attr, batch, W_edge, eps, W1, b1, W2, b2,
           Wv1, bv1, Wv2, bv2, vn0, Wpred, bpred):
    x_pad = jnp.zeros((NP, D), _f32).at[:N].set(x)
    batch_pad = jnp.full((1, NP), G, jnp.int32).at[0, :N].set(batch)
    src_pad = (jnp.zeros((EP,), jnp.int32).at[:E].set(edge_index[0])
               .reshape(EP // CH, CH))
    dst_pad = (jnp.full((EP,), N, jnp.int32).at[:E].set(edge_index[1])
               .reshape(EP // CH, CH))
    ea_pad = jnp.zeros((EP, DE), _f32).at[:E].set(edge_attr)
    zeros2d = jnp.zeros((RPT, D), _f32)
    vn = jnp.broadcast_to(vn0, (G, D)) + jnp.zeros((G, D), _f32)

    emb_all = _emb_call(ea_pad, W_edge)

    h = x_pad
    for l in range(L):
        lv = min(l, L - 2)
        hl, vn_next = _layer_in_call(h, batch_pad, vn, Wv1[lv],
                                     bv1[lv].reshape(1, 2 * D), Wv2[lv],
                                     bv2[lv].reshape(1, D))
        agg = _sc_calls[l](hl, src_pad, dst_pad, emb_all, zeros2d)
        call = _layer_out_final_call if l == L - 1 else _layer_out_call
        h = call(hl, agg, eps[l].reshape(1, 1), W1[l], b1[l].reshape(1, 2 * D),
                 W2[l], b2[l].reshape(1, D))
        vn = vn_next

    return _head_call(h, batch_pad, Wpred, bpred.reshape(1, NCLS))
